# Initial kernel scaffold; baseline (speedup 1.0000x reference)
import jax
import jax.numpy as jnp
from jax.experimental import pallas as pl


def kernel(x, edge_index, batch, params):
    raise NotImplementedError("write your pallas kernel here")


# speedup vs baseline: 13.9261x; 13.9261x over previous
import functools

import jax
import jax.numpy as jnp
from jax import lax
from jax.experimental import pallas as pl
from jax.experimental.pallas import tpu as pltpu
from jax.experimental.pallas import tpu_sc as plsc

N = 10000
NP = 10240
E = 320000
G = 128
D = 128
DE = 144
R = 1024
NBLK = NP // R
NC, NS = 2, 16
NW = NC * NS
EPW = E // NW
SUP = 160
NSUP = E // SUP
EPS = 1e-5

_PC = pl.pallas_call


@functools.cache
def _mesh():
    return plsc.VectorSubcoreMesh(
        core_axis_name="c", subcore_axis_name="s",
        num_cores=NC, num_subcores=NS)


@functools.cache
def _make_sc_emb_gather():
    @functools.partial(
        pl.kernel,
        out_type=jax.ShapeDtypeStruct((NP, D), jnp.float32),
        mesh=_mesh(),
        scratch_types=[
            pltpu.VMEM((NP // NW,), jnp.int32),
            pltpu.VMEM((NP // NW, D), jnp.float32),
            pltpu.SemaphoreType.DMA,
        ],
    )
    def body(emb_hbm, idx_hbm, out_hbm, idx_v, rows_v, sem):
        wid = lax.axis_index("s") * NC + lax.axis_index("c")
        bpw = NP // NW
        base = pl.multiple_of(wid * bpw, bpw)
        pltpu.sync_copy(idx_hbm.at[pl.ds(base, bpw)], idx_v)
        descs = [
            pltpu.async_copy(
                emb_hbm.at[idx_v.at[pl.ds(j * 80, 80)]],
                rows_v.at[pl.ds(j * 80, 80), :], sem)
            for j in range(bpw // 80)
        ]
        for d in descs:
            d.wait()
        pltpu.sync_copy(rows_v, out_hbm.at[pl.ds(base, bpw), :])

    return body


def _sc_emb_gather(emb, idxp):
    return _make_sc_emb_gather()(emb, idxp)


@functools.cache
def _make_sc_edge_agg():
    @functools.partial(
        pl.kernel,
        out_type=jax.ShapeDtypeStruct((NC, NP, DE), jnp.float32),
        mesh=_mesh(),
        compiler_params=pltpu.CompilerParams(use_tc_tiling_on_sc=False),
        scratch_types=[
            pltpu.VMEM_SHARED((NP, DE), jnp.float32),
            pltpu.VMEM((SUP,), jnp.int32),
            pltpu.VMEM((SUP // 80, 80), jnp.int32),
            pltpu.VMEM((SUP, DE), jnp.float32),
            pltpu.SemaphoreType.DMA,
        ],
    )
    def body(xe_hbm, src_hbm, dst_hbm, z_hbm, agg_hbm,
             shared, se_v, de_v, rows_v, sem):
        c = lax.axis_index("c")
        s = lax.axis_index("s")
        wid = s * NC + c
        rps = NP // NS
        rbase = pl.multiple_of(s * rps, rps)
        pltpu.sync_copy(z_hbm, shared.at[pl.ds(rbase, rps), :])
        plsc.subcore_barrier()

        def do_super(sid):
            ebase = pl.multiple_of(sid * SUP, SUP)
            pltpu.sync_copy(src_hbm.at[pl.ds(ebase, SUP)], se_v)
            for b in range(SUP // 80):
                pltpu.sync_copy(dst_hbm.at[pl.ds(ebase + b * 80, 80)],
                                de_v.at[b])
            descs = [
                pltpu.async_copy(
                    xe_hbm.at[se_v.at[pl.ds(b * 80, 80)]],
                    rows_v.at[pl.ds(b * 80, 80), :], sem)
                for b in range(SUP // 80)
            ]
            for d in descs:
                d.wait()
            for b in range(SUP // 80):
                pltpu.sync_copy(rows_v.at[pl.ds(b * 80, 80), :],
                                shared.at[de_v.at[b]], add=True)

        def loop_body(t, carry):
            sid = t * NW + wid

            @pl.when(sid < NSUP)
            def _():
                do_super(sid)

            return carry

        lax.fori_loop(0, (NSUP + NW - 1) // NW, loop_body, 0)
        plsc.subcore_barrier()
        pltpu.sync_copy(shared.at[pl.ds(rbase, rps), :],
                        agg_hbm.at[c, pl.ds(rbase, rps), :])

    return body


def _sc_edge_agg(xe, src, dst, zrows):
    return _make_sc_edge_agg()(xe, src, dst, zrows)


def _t1_body(agg_ref, x_ref, k_ref, b_ref, wl_ref, bl_ref, wr_ref,
             attn_ref, h_ref, s_ref, cnt_ref):
    a2 = agg_ref[...]
    agg = a2[0] + a2[1]
    deg = agg[:, D:D + 1]
    aggn = agg[:, :D] / jnp.clip(deg, 1.0, None)
    h = aggn @ wl_ref[...] + bl_ref[...] + x_ref[...] @ wr_ref[...]
    h = jnp.maximum(h, 0.0)
    h_ref[...] = h
    attn = attn_ref[...]
    nrm = jnp.sqrt(jnp.sum(attn * attn))
    s_ref[...] = jnp.tanh(jnp.sum(h * attn, axis=1, keepdims=True) / nrm)
    gid = lax.broadcasted_iota(jnp.int32, (1, G), 1)
    oh = jnp.where((b_ref[...] == gid) & (k_ref[...] > 0.0), 1.0, 0.0)
    cnt = jnp.sum(oh, axis=0, keepdims=True)

    @pl.when(pl.program_id(0) == 0)
    def _():
        cnt_ref[...] = jnp.zeros_like(cnt_ref)

    cnt_ref[...] += cnt


def _t1_sage(agg2, x, kv, bc, wl, bl, wr, attn):
    return _PC(
        _t1_body,
        grid=(NBLK,),
        in_specs=[
            pl.BlockSpec((NC, R, DE), lambda i: (0, i, 0)),
            pl.BlockSpec((R, D), lambda i: (i, 0)),
            pl.BlockSpec((R, 1), lambda i: (i, 0)),
            pl.BlockSpec((R, 1), lambda i: (i, 0)),
            pl.BlockSpec((D, D), lambda i: (0, 0)),
            pl.BlockSpec((1, D), lambda i: (0, 0)),
            pl.BlockSpec((D, D), lambda i: (0, 0)),
            pl.BlockSpec((1, D), lambda i: (0, 0)),
        ],
        out_specs=[
            pl.BlockSpec((R, D), lambda i: (i, 0)),
            pl.BlockSpec((R, 1), lambda i: (i, 0)),
            pl.BlockSpec((1, G), lambda i: (0, 0)),
        ],
        out_shape=[
            jax.ShapeDtypeStruct((NP, D), jnp.float32),
            jax.ShapeDtypeStruct((NP, 1), jnp.float32),
            jax.ShapeDtypeStruct((1, G), jnp.float32),
        ],
    )(agg2, x, kv, bc, wl, bl, wr, attn)


def _t2_body(si_ref, bi_ref, ki_ref, sr_ref, br_ref, kr_ref, kq_ref,
             bmin_ref, bmax_ref, keep_ref):
    p = pl.program_id(0)
    si = si_ref[...]
    bi = bi_ref[...]
    vi = ki_ref[...] > 0.0
    idxi = p * R + lax.broadcasted_iota(jnp.int32, (R, 1), 0)
    gmin = bmin_ref[p * (R // G)]
    gmax = bmax_ref[p * (R // G) + (R // G) - 1]

    def jbody(j, acc):
        def hit():
            sj = sr_ref[pl.ds(j, 1), :]
            bj = br_ref[pl.ds(j, 1), :]
            vj = kr_ref[pl.ds(j, 1), :] > 0.0
            idxj = j * G + lax.broadcasted_iota(jnp.int32, (1, G), 1)
            higher = (sj > si) | ((sj == si) & (idxj < idxi))
            m = (bj == bi) & vj & higher
            return acc + jnp.sum(jnp.where(m, 1.0, 0.0), axis=1, keepdims=True)

        pred = (bmax_ref[j] >= gmin) & (bmin_ref[j] <= gmax)
        return lax.cond(pred, hit, lambda: acc)

    rank = lax.fori_loop(0, NP // G, jbody, jnp.zeros((R, 1), jnp.float32))
    gid = lax.broadcasted_iota(jnp.int32, (1, G), 1)
    oh = jnp.where(bi == gid, 1.0, 0.0)
    kq = jnp.sum(oh * kq_ref[...], axis=1, keepdims=True)
    keep_ref[...] = jnp.where(vi & (rank < kq), 1.0, 0.0)


def _t2_rank(score, bc, kv, sr, br, kr, kq, bsmin, bsmax):
    return _PC(
        _t2_body,
        grid=(NBLK,),
        in_specs=[
            pl.BlockSpec((R, 1), lambda i: (i, 0)),
            pl.BlockSpec((R, 1), lambda i: (i, 0)),
            pl.BlockSpec((R, 1), lambda i: (i, 0)),
            pl.BlockSpec((NP // G, G), lambda i: (0, 0)),
            pl.BlockSpec((NP // G, G), lambda i: (0, 0)),
            pl.BlockSpec((NP // G, G), lambda i: (0, 0)),
            pl.BlockSpec((1, G), lambda i: (0, 0)),
            pl.BlockSpec(memory_space=pltpu.SMEM),
            pl.BlockSpec(memory_space=pltpu.SMEM),
        ],
        out_specs=pl.BlockSpec((R, 1), lambda i: (i, 0)),
        out_shape=jax.ShapeDtypeStruct((NP, 1), jnp.float32),
    )(score, bc, kv, sr, br, kr, kq, bsmin, bsmax)


def _t3_body(h_ref, s_ref, kp_ref, b_ref, glo_ref, ghi_ref,
             mx_ref, sm_ref, cnt_ref, xs_ref, xe_ref):
    p = pl.program_id(0)

    @pl.when(p == 0)
    def _():
        mx_ref[...] = jnp.full_like(mx_ref, -jnp.inf)
        sm_ref[...] = jnp.zeros_like(sm_ref)
        cnt_ref[...] = jnp.zeros_like(cnt_ref)

    xs = h_ref[...] * s_ref[...]
    xs_ref[...] = xs
    kp = kp_ref[...]
    xe_ref[...] = jnp.concatenate(
        [xs * kp, kp, jnp.zeros((R, DE - D - 1), jnp.float32)], axis=1)
    b = b_ref[...]

    def gbody(g, carry):
        m = (b == g) & (kp > 0.0)
        mf = jnp.where(m, 1.0, 0.0)
        xm = jnp.where(m, xs, -jnp.inf)
        mx_ref[pl.ds(g, 1), :] = jnp.maximum(
            mx_ref[pl.ds(g, 1), :], jnp.max(xm, axis=0, keepdims=True))
        sm_ref[pl.ds(g, 1), :] += jnp.sum(xs * mf, axis=0, keepdims=True)
        cnt_ref[pl.ds(g, 1), :] += jnp.sum(mf).reshape(1, 1)
        return carry

    lax.fori_loop(glo_ref[p], ghi_ref[p] + 1, gbody, 0)


def _t3_readout(h, score, keep, bc, gblo, gbhi):
    return _PC(
        _t3_body,
        grid=(NBLK,),
        in_specs=[
            pl.BlockSpec((R, D), lambda i: (i, 0)),
            pl.BlockSpec((R, 1), lambda i: (i, 0)),
            pl.BlockSpec((R, 1), lambda i: (i, 0)),
            pl.BlockSpec((R, 1), lambda i: (i, 0)),
            pl.BlockSpec(memory_space=pltpu.SMEM),
            pl.BlockSpec(memory_space=pltpu.SMEM),
        ],
        out_specs=[
            pl.BlockSpec((G, D), lambda i: (0, 0)),
            pl.BlockSpec((G, D), lambda i: (0, 0)),
            pl.BlockSpec((G, 1), lambda i: (0, 0)),
            pl.BlockSpec((R, D), lambda i: (i, 0)),
            pl.BlockSpec((R, DE), lambda i: (i, 0)),
        ],
        out_shape=[
            jax.ShapeDtypeStruct((G, D), jnp.float32),
            jax.ShapeDtypeStruct((G, D), jnp.float32),
            jax.ShapeDtypeStruct((G, 1), jnp.float32),
            jax.ShapeDtypeStruct((NP, D), jnp.float32),
            jax.ShapeDtypeStruct((NP, DE), jnp.float32),
        ],
    )(h, score, keep, bc, gblo, gbhi)


def _t4_body(mx1, sm1, c1, mx2, sm2, c2, mx3, sm3, c3,
             wf1, bf1, g1, be1, wf2, bf2, g2, be2, wo, bo, out_ref):
    def readout(mx_ref, sm_ref, c_ref):
        cnt = c_ref[...]
        mx = jnp.where(cnt > 0.0, mx_ref[...], 0.0)
        mn = sm_ref[...] / jnp.clip(cnt, 1.0, None)
        return jnp.concatenate([mx, mn], axis=1)

    def bn(x, g_ref, be_ref):
        mu = jnp.mean(x, axis=0, keepdims=True)
        xc = x - mu
        var = jnp.mean(xc * xc, axis=0, keepdims=True)
        return g_ref[...] * xc / jnp.sqrt(var + EPS) + be_ref[...]

    h = (readout(mx1, sm1, c1) + readout(mx2, sm2, c2)
         + readout(mx3, sm3, c3))
    h = jnp.maximum(bn(h @ wf1[...] + bf1[...], g1, be1), 0.0)
    h = jnp.maximum(bn(h @ wf2[...] + bf2[...], g2, be2), 0.0)
    z = jnp.sum(h * wo[...], axis=1, keepdims=True) + bo[...]
    out_ref[...] = 1.0 / (1.0 + jnp.exp(-z))


def _t4_head(reads, p):
    args = []
    for mx, sm, cnt in reads:
        args += [mx, sm, cnt]
    args += [
        p['W_fc1'], p['b_fc1'].reshape(1, D), p['g1'].reshape(1, D),
        p['be1'].reshape(1, D),
        p['W_fc2'], p['b_fc2'].reshape(1, 64), p['g2'].reshape(1, 64),
        p['be2'].reshape(1, 64),
        p['W_out'].reshape(1, 64), p['b_out'].reshape(1, 1),
    ]
    return _PC(
        _t4_body,
        out_shape=jax.ShapeDtypeStruct((G, 1), jnp.float32),
    )(*args)


def kernel(x, edge_index, batch, params):
    p = params
    idxp = jnp.concatenate(
        [x[:, 0], jnp.zeros((NP - N,), jnp.int32)])
    bp = jnp.concatenate(
        [batch, jnp.full((NP - N,), G - 1, jnp.int32)])
    bc = bp.reshape(NP, 1)
    br = bp.reshape(NP // G, G)
    bsmin = br[:, 0]
    bsmax = br[:, -1]
    gblo = bp[::R]
    gbhi = bp[R - 1::R]
    src = edge_index[0]
    dst = edge_index[1]
    zrows = jnp.zeros((NP // NS, DE), jnp.float32)
    kv = jnp.concatenate(
        [jnp.ones((N,), jnp.float32),
         jnp.zeros((NP - N,), jnp.float32)]).reshape(NP, 1)

    xc = _sc_emb_gather(p['emb'], idxp)
    xe = jnp.concatenate(
        [xc * kv, kv, jnp.zeros((NP, DE - D - 1), jnp.float32)], axis=1)

    reads = []
    for c in (1, 2, 3):
        agg2 = _sc_edge_agg(xe, src, dst, zrows)
        h, score, counts = _t1_sage(
            agg2, xc, kv, bc, p['Wl%d' % c], p['bl%d' % c].reshape(1, D),
            p['Wr%d' % c], p['attn%d' % c].reshape(1, D))
        kq = ((4 * counts.astype(jnp.int32) + 4) // 5).astype(jnp.float32)
        keep = _t2_rank(score, bc, kv, score.reshape(NP // G, G), br,
                        kv.reshape(NP // G, G), kq, bsmin, bsmax)
        mx, sm, cnt, xs, xe = _t3_readout(h, score, keep, bc, gblo, gbhi)
        reads.append((mx, sm, cnt))
        xc, kv = xs, keep

    return _t4_head(reads, p).reshape(G)


# speedup vs baseline: 16.4022x; 1.1778x over previous
import functools

import jax
import jax.numpy as jnp
from jax import lax
from jax.experimental import pallas as pl
from jax.experimental.pallas import tpu as pltpu
from jax.experimental.pallas import tpu_sc as plsc

N = 10000
NP = 10240
E = 320000
G = 128
D = 128
DE = 144
R = 1024
NBLK = NP // R
NC, NS = 2, 16
NW = NC * NS
EPW = E // NW
SUP = 640
NSUP = E // SUP
RB = 3
EPS = 1e-5

_PC = pl.pallas_call


@functools.cache
def _mesh():
    return plsc.VectorSubcoreMesh(
        core_axis_name="c", subcore_axis_name="s",
        num_cores=NC, num_subcores=NS)


@functools.cache
def _make_sc_emb_gather():
    @functools.partial(
        pl.kernel,
        out_type=jax.ShapeDtypeStruct((NP, D), jnp.float32),
        mesh=_mesh(),
        scratch_types=[
            pltpu.VMEM((NP // NW,), jnp.int32),
            pltpu.VMEM((NP // NW, D), jnp.float32),
            pltpu.SemaphoreType.DMA,
        ],
    )
    def body(emb_hbm, idx_hbm, out_hbm, idx_v, rows_v, sem):
        wid = lax.axis_index("s") * NC + lax.axis_index("c")
        bpw = NP // NW
        base = pl.multiple_of(wid * bpw, bpw)
        pltpu.sync_copy(idx_hbm.at[pl.ds(base, bpw)], idx_v)
        descs = [
            pltpu.async_copy(
                emb_hbm.at[idx_v.at[pl.ds(j * 80, 80)]],
                rows_v.at[pl.ds(j * 80, 80), :], sem)
            for j in range(bpw // 80)
        ]
        for d in descs:
            d.wait()
        pltpu.sync_copy(rows_v, out_hbm.at[pl.ds(base, bpw), :])

    return body


def _sc_emb_gather(emb, idxp):
    return _make_sc_emb_gather()(emb, idxp)


@functools.cache
def _make_sc_edge_agg():
    @functools.partial(
        pl.kernel,
        out_type=jax.ShapeDtypeStruct((NC, NP, DE), jnp.float32),
        mesh=_mesh(),
        compiler_params=pltpu.CompilerParams(use_tc_tiling_on_sc=False),
        scratch_types=[
            pltpu.VMEM_SHARED((NP, DE), jnp.float32),
            pltpu.VMEM((SUP,), jnp.int32),
            pltpu.VMEM((SUP // 80, 80), jnp.int32),
            pltpu.VMEM((RB, 80, DE), jnp.float32),
            pltpu.SemaphoreType.DMA,
            pltpu.SemaphoreType.DMA,
        ],
    )
    def body(xe_hbm, src_hbm, dst_hbm, z_hbm, agg_hbm,
             shared, se_v, de_v, rows_v, sem_g, sem_s):
        c = lax.axis_index("c")
        s = lax.axis_index("s")
        wid = s * NC + c
        rps = NP // NS
        rbase = pl.multiple_of(s * rps, rps)
        pltpu.sync_copy(z_hbm, shared.at[pl.ds(rbase, rps), :])
        plsc.subcore_barrier()
        nch = SUP // 80

        def do_super(sid):
            ebase = pl.multiple_of(sid * SUP, SUP)
            pltpu.sync_copy(src_hbm.at[pl.ds(ebase, SUP)], se_v)
            for b in range(nch):
                pltpu.sync_copy(dst_hbm.at[pl.ds(ebase + b * 80, 80)],
                                de_v.at[b])

            def gather(b):
                return pltpu.async_copy(
                    xe_hbm.at[se_v.at[pl.ds(b * 80, 80)]],
                    rows_v.at[b % RB], sem_g)

            gd = {b: gather(b) for b in range(RB)}
            sd = {}
            for b in range(nch):
                gd[b].wait()
                sd[b] = pltpu.async_copy(
                    rows_v.at[b % RB], shared.at[de_v.at[b]], sem_s,
                    add=True)
                if b - (RB - 1) >= 0:
                    sd[b - (RB - 1)].wait()
                if b + RB < nch:
                    gd[b + RB] = gather(b + RB)
            for b in range(nch - (RB - 1), nch):
                if b >= 0:
                    sd[b].wait()

        def loop_body(t, carry):
            sid = t * NW + wid

            @pl.when(sid < NSUP)
            def _():
                do_super(sid)

            return carry

        lax.fori_loop(0, (NSUP + NW - 1) // NW, loop_body, 0)
        plsc.subcore_barrier()
        pltpu.sync_copy(shared.at[pl.ds(rbase, rps), :],
                        agg_hbm.at[c, pl.ds(rbase, rps), :])

    return body


def _sc_edge_agg(xe, src, dst, zrows):
    return _make_sc_edge_agg()(xe, src, dst, zrows)


def _t1_body(agg_ref, x_ref, k_ref, b_ref, wl_ref, bl_ref, wr_ref,
             attn_ref, h_ref, s_ref, cnt_ref):
    a2 = agg_ref[...]
    agg = a2[0] + a2[1]
    deg = agg[:, D:D + 1]
    aggn = agg[:, :D] / jnp.clip(deg, 1.0, None)
    h = aggn @ wl_ref[...] + bl_ref[...] + x_ref[...] @ wr_ref[...]
    h = jnp.maximum(h, 0.0)
    h_ref[...] = h
    attn = attn_ref[...]
    nrm = jnp.sqrt(jnp.sum(attn * attn))
    s_ref[...] = jnp.tanh(jnp.sum(h * attn, axis=1, keepdims=True) / nrm)
    gid = lax.broadcasted_iota(jnp.int32, (1, G), 1)
    oh = jnp.where((b_ref[...] == gid) & (k_ref[...] > 0.0), 1.0, 0.0)
    cnt = jnp.sum(oh, axis=0, keepdims=True)

    @pl.when(pl.program_id(0) == 0)
    def _():
        cnt_ref[...] = jnp.zeros_like(cnt_ref)

    cnt_ref[...] += cnt


def _t1_sage(agg2, x, kv, bc, wl, bl, wr, attn):
    return _PC(
        _t1_body,
        grid=(NBLK,),
        in_specs=[
            pl.BlockSpec((NC, R, DE), lambda i: (0, i, 0)),
            pl.BlockSpec((R, D), lambda i: (i, 0)),
            pl.BlockSpec((R, 1), lambda i: (i, 0)),
            pl.BlockSpec((R, 1), lambda i: (i, 0)),
            pl.BlockSpec((D, D), lambda i: (0, 0)),
            pl.BlockSpec((1, D), lambda i: (0, 0)),
            pl.BlockSpec((D, D), lambda i: (0, 0)),
            pl.BlockSpec((1, D), lambda i: (0, 0)),
        ],
        out_specs=[
            pl.BlockSpec((R, D), lambda i: (i, 0)),
            pl.BlockSpec((R, 1), lambda i: (i, 0)),
            pl.BlockSpec((1, G), lambda i: (0, 0)),
        ],
        out_shape=[
            jax.ShapeDtypeStruct((NP, D), jnp.float32),
            jax.ShapeDtypeStruct((NP, 1), jnp.float32),
            jax.ShapeDtypeStruct((1, G), jnp.float32),
        ],
    )(agg2, x, kv, bc, wl, bl, wr, attn)


def _t2_body(si_ref, bi_ref, ki_ref, sr_ref, br_ref, kr_ref, kq_ref,
             bmin_ref, bmax_ref, keep_ref):
    p = pl.program_id(0)
    si = si_ref[...]
    bi = bi_ref[...]
    vi = ki_ref[...] > 0.0
    idxi = p * R + lax.broadcasted_iota(jnp.int32, (R, 1), 0)
    gmin = bmin_ref[p * (R // G)]
    gmax = bmax_ref[p * (R // G) + (R // G) - 1]

    def jbody(j, acc):
        def hit():
            sj = sr_ref[pl.ds(j, 1), :]
            bj = br_ref[pl.ds(j, 1), :]
            vj = kr_ref[pl.ds(j, 1), :] > 0.0
            idxj = j * G + lax.broadcasted_iota(jnp.int32, (1, G), 1)
            higher = (sj > si) | ((sj == si) & (idxj < idxi))
            m = (bj == bi) & vj & higher
            return acc + jnp.sum(jnp.where(m, 1.0, 0.0), axis=1, keepdims=True)

        pred = (bmax_ref[j] >= gmin) & (bmin_ref[j] <= gmax)
        return lax.cond(pred, hit, lambda: acc)

    rank = lax.fori_loop(0, NP // G, jbody, jnp.zeros((R, 1), jnp.float32))
    gid = lax.broadcasted_iota(jnp.int32, (1, G), 1)
    oh = jnp.where(bi == gid, 1.0, 0.0)
    kq = jnp.sum(oh * kq_ref[...], axis=1, keepdims=True)
    keep_ref[...] = jnp.where(vi & (rank < kq), 1.0, 0.0)


def _t2_rank(score, bc, kv, sr, br, kr, kq, bsmin, bsmax):
    return _PC(
        _t2_body,
        grid=(NBLK,),
        in_specs=[
            pl.BlockSpec((R, 1), lambda i: (i, 0)),
            pl.BlockSpec((R, 1), lambda i: (i, 0)),
            pl.BlockSpec((R, 1), lambda i: (i, 0)),
            pl.BlockSpec((NP // G, G), lambda i: (0, 0)),
            pl.BlockSpec((NP // G, G), lambda i: (0, 0)),
            pl.BlockSpec((NP // G, G), lambda i: (0, 0)),
            pl.BlockSpec((1, G), lambda i: (0, 0)),
            pl.BlockSpec(memory_space=pltpu.SMEM),
            pl.BlockSpec(memory_space=pltpu.SMEM),
        ],
        out_specs=pl.BlockSpec((R, 1), lambda i: (i, 0)),
        out_shape=jax.ShapeDtypeStruct((NP, 1), jnp.float32),
    )(score, bc, kv, sr, br, kr, kq, bsmin, bsmax)


def _t3_body(h_ref, s_ref, kp_ref, b_ref, glo_ref, ghi_ref,
             mx_ref, sm_ref, cnt_ref, xs_ref, xe_ref):
    p = pl.program_id(0)

    @pl.when(p == 0)
    def _():
        mx_ref[...] = jnp.full_like(mx_ref, -jnp.inf)
        sm_ref[...] = jnp.zeros_like(sm_ref)
        cnt_ref[...] = jnp.zeros_like(cnt_ref)

    xs = h_ref[...] * s_ref[...]
    xs_ref[...] = xs
    kp = kp_ref[...]
    xe_ref[...] = jnp.concatenate(
        [xs * kp, kp, jnp.zeros((R, DE - D - 1), jnp.float32)], axis=1)
    b = b_ref[...]

    def gbody(g, carry):
        m = (b == g) & (kp > 0.0)
        mf = jnp.where(m, 1.0, 0.0)
        xm = jnp.where(m, xs, -jnp.inf)
        mx_ref[pl.ds(g, 1), :] = jnp.maximum(
            mx_ref[pl.ds(g, 1), :], jnp.max(xm, axis=0, keepdims=True))
        sm_ref[pl.ds(g, 1), :] += jnp.sum(xs * mf, axis=0, keepdims=True)
        cnt_ref[pl.ds(g, 1), :] += jnp.sum(mf).reshape(1, 1)
        return carry

    lax.fori_loop(glo_ref[p], ghi_ref[p] + 1, gbody, 0)


def _t3_readout(h, score, keep, bc, gblo, gbhi):
    return _PC(
        _t3_body,
        grid=(NBLK,),
        in_specs=[
            pl.BlockSpec((R, D), lambda i: (i, 0)),
            pl.BlockSpec((R, 1), lambda i: (i, 0)),
            pl.BlockSpec((R, 1), lambda i: (i, 0)),
            pl.BlockSpec((R, 1), lambda i: (i, 0)),
            pl.BlockSpec(memory_space=pltpu.SMEM),
            pl.BlockSpec(memory_space=pltpu.SMEM),
        ],
        out_specs=[
            pl.BlockSpec((G, D), lambda i: (0, 0)),
            pl.BlockSpec((G, D), lambda i: (0, 0)),
            pl.BlockSpec((G, 1), lambda i: (0, 0)),
            pl.BlockSpec((R, D), lambda i: (i, 0)),
            pl.BlockSpec((R, DE), lambda i: (i, 0)),
        ],
        out_shape=[
            jax.ShapeDtypeStruct((G, D), jnp.float32),
            jax.ShapeDtypeStruct((G, D), jnp.float32),
            jax.ShapeDtypeStruct((G, 1), jnp.float32),
            jax.ShapeDtypeStruct((NP, D), jnp.float32),
            jax.ShapeDtypeStruct((NP, DE), jnp.float32),
        ],
    )(h, score, keep, bc, gblo, gbhi)


def _t4_body(mx1, sm1, c1, mx2, sm2, c2, mx3, sm3, c3,
             wf1, bf1, g1, be1, wf2, bf2, g2, be2, wo, bo, out_ref):
    def readout(mx_ref, sm_ref, c_ref):
        cnt = c_ref[...]
        mx = jnp.where(cnt > 0.0, mx_ref[...], 0.0)
        mn = sm_ref[...] / jnp.clip(cnt, 1.0, None)
        return jnp.concatenate([mx, mn], axis=1)

    def bn(x, g_ref, be_ref):
        mu = jnp.mean(x, axis=0, keepdims=True)
        xc = x - mu
        var = jnp.mean(xc * xc, axis=0, keepdims=True)
        return g_ref[...] * xc / jnp.sqrt(var + EPS) + be_ref[...]

    h = (readout(mx1, sm1, c1) + readout(mx2, sm2, c2)
         + readout(mx3, sm3, c3))
    h = jnp.maximum(bn(h @ wf1[...] + bf1[...], g1, be1), 0.0)
    h = jnp.maximum(bn(h @ wf2[...] + bf2[...], g2, be2), 0.0)
    z = jnp.sum(h * wo[...], axis=1, keepdims=True) + bo[...]
    out_ref[...] = 1.0 / (1.0 + jnp.exp(-z))


def _t4_head(reads, p):
    args = []
    for mx, sm, cnt in reads:
        args += [mx, sm, cnt]
    args += [
        p['W_fc1'], p['b_fc1'].reshape(1, D), p['g1'].reshape(1, D),
        p['be1'].reshape(1, D),
        p['W_fc2'], p['b_fc2'].reshape(1, 64), p['g2'].reshape(1, 64),
        p['be2'].reshape(1, 64),
        p['W_out'].reshape(1, 64), p['b_out'].reshape(1, 1),
    ]
    return _PC(
        _t4_body,
        out_shape=jax.ShapeDtypeStruct((G, 1), jnp.float32),
    )(*args)


def kernel(x, edge_index, batch, params):
    p = params
    idxp = jnp.concatenate(
        [x[:, 0], jnp.zeros((NP - N,), jnp.int32)])
    bp = jnp.concatenate(
        [batch, jnp.full((NP - N,), G - 1, jnp.int32)])
    bc = bp.reshape(NP, 1)
    br = bp.reshape(NP // G, G)
    bsmin = br[:, 0]
    bsmax = br[:, -1]
    gblo = bp[::R]
    gbhi = bp[R - 1::R]
    src = edge_index[0]
    dst = edge_index[1]
    zrows = jnp.zeros((NP // NS, DE), jnp.float32)
    kv = jnp.concatenate(
        [jnp.ones((N,), jnp.float32),
         jnp.zeros((NP - N,), jnp.float32)]).reshape(NP, 1)

    xc = _sc_emb_gather(p['emb'], idxp)
    xe = jnp.concatenate(
        [xc * kv, kv, jnp.zeros((NP, DE - D - 1), jnp.float32)], axis=1)

    reads = []
    for c in (1, 2, 3):
        agg2 = _sc_edge_agg(xe, src, dst, zrows)
        h, score, counts = _t1_sage(
            agg2, xc, kv, bc, p['Wl%d' % c], p['bl%d' % c].reshape(1, D),
            p['Wr%d' % c], p['attn%d' % c].reshape(1, D))
        kq = ((4 * counts.astype(jnp.int32) + 4) // 5).astype(jnp.float32)
        keep = _t2_rank(score, bc, kv, score.reshape(NP // G, G), br,
                        kv.reshape(NP // G, G), kq, bsmin, bsmax)
        mx, sm, cnt, xs, xe = _t3_readout(h, score, keep, bc, gblo, gbhi)
        reads.append((mx, sm, cnt))
        xc, kv = xs, keep

    return _t4_head(reads, p).reshape(G)


# speedup vs baseline: 16.5403x; 1.0084x over previous
import functools

import jax
import jax.numpy as jnp
from jax import lax
from jax.experimental import pallas as pl
from jax.experimental.pallas import tpu as pltpu
from jax.experimental.pallas import tpu_sc as plsc

N = 10000
NP = 10240
E = 320000
G = 128
D = 128
DE = 144
R = 1024
NBLK = NP // R
NC, NS = 2, 16
NW = NC * NS
EPW = E // NW
SUP = 640
NSUP = E // SUP
RB = 3
EPS = 1e-5

_PC = pl.pallas_call


@functools.cache
def _mesh():
    return plsc.VectorSubcoreMesh(
        core_axis_name="c", subcore_axis_name="s",
        num_cores=NC, num_subcores=NS)


@functools.cache
def _make_sc_emb_gather():
    @functools.partial(
        pl.kernel,
        out_type=jax.ShapeDtypeStruct((NP, D), jnp.float32),
        mesh=_mesh(),
        scratch_types=[
            pltpu.VMEM((NP // NW,), jnp.int32),
            pltpu.VMEM((NP // NW, D), jnp.float32),
            pltpu.SemaphoreType.DMA,
        ],
    )
    def body(emb_hbm, idx_hbm, out_hbm, idx_v, rows_v, sem):
        wid = lax.axis_index("s") * NC + lax.axis_index("c")
        bpw = NP // NW
        base = pl.multiple_of(wid * bpw, bpw)
        pltpu.sync_copy(idx_hbm.at[pl.ds(base, bpw)], idx_v)
        descs = [
            pltpu.async_copy(
                emb_hbm.at[idx_v.at[pl.ds(j * 80, 80)]],
                rows_v.at[pl.ds(j * 80, 80), :], sem)
            for j in range(bpw // 80)
        ]
        for d in descs:
            d.wait()
        pltpu.sync_copy(rows_v, out_hbm.at[pl.ds(base, bpw), :])

    return body


def _sc_emb_gather(emb, idxp):
    return _make_sc_emb_gather()(emb, idxp)


@functools.cache
def _make_sc_edge_agg():
    @functools.partial(
        pl.kernel,
        out_type=jax.ShapeDtypeStruct((NC, NP, DE), jnp.float32),
        mesh=_mesh(),
        compiler_params=pltpu.CompilerParams(use_tc_tiling_on_sc=False),
        scratch_types=[
            pltpu.VMEM_SHARED((NP, DE), jnp.float32),
            pltpu.VMEM((SUP,), jnp.int32),
            pltpu.VMEM((SUP // 80, 80), jnp.int32),
            pltpu.VMEM((RB, 80, DE), jnp.float32),
            pltpu.SemaphoreType.DMA,
            pltpu.SemaphoreType.DMA,
        ],
    )
    def body(xe_hbm, src_hbm, dst_hbm, z_hbm, agg_hbm,
             shared, se_v, de_v, rows_v, sem_g, sem_s):
        c = lax.axis_index("c")
        s = lax.axis_index("s")
        wid = s * NC + c
        rps = NP // NS
        rbase = pl.multiple_of(s * rps, rps)
        pltpu.sync_copy(z_hbm, shared.at[pl.ds(rbase, rps), :])
        plsc.subcore_barrier()
        nch = SUP // 80

        def do_super(sid):
            ebase = pl.multiple_of(sid * SUP, SUP)
            pltpu.sync_copy(src_hbm.at[pl.ds(ebase, SUP)], se_v)
            for b in range(nch):
                pltpu.sync_copy(dst_hbm.at[pl.ds(ebase + b * 80, 80)],
                                de_v.at[b])

            def gather(b):
                return pltpu.async_copy(
                    xe_hbm.at[se_v.at[pl.ds(b * 80, 80)]],
                    rows_v.at[b % RB], sem_g)

            gd = {b: gather(b) for b in range(RB)}
            sd = {}
            for b in range(nch):
                gd[b].wait()
                sd[b] = pltpu.async_copy(
                    rows_v.at[b % RB], shared.at[de_v.at[b]], sem_s,
                    add=True)
                if b - (RB - 1) >= 0:
                    sd[b - (RB - 1)].wait()
                if b + RB < nch:
                    gd[b + RB] = gather(b + RB)
            for b in range(nch - (RB - 1), nch):
                if b >= 0:
                    sd[b].wait()

        def loop_body(t, carry):
            sid = t * NW + wid

            @pl.when(sid < NSUP)
            def _():
                do_super(sid)

            return carry

        lax.fori_loop(0, (NSUP + NW - 1) // NW, loop_body, 0)
        plsc.subcore_barrier()
        pltpu.sync_copy(shared.at[pl.ds(rbase, rps), :],
                        agg_hbm.at[c, pl.ds(rbase, rps), :])

    return body


def _sc_edge_agg(xe, src, dst, zrows):
    return _make_sc_edge_agg()(xe, src, dst, zrows)


def _t1_body(agg_ref, x_ref, k_ref, b_ref, wl_ref, bl_ref, wr_ref,
             attn_ref, h_ref, s_ref, cnt_ref):
    a2 = agg_ref[...]
    agg = a2[0] + a2[1]
    deg = agg[:, D:D + 1]
    aggn = agg[:, :D] / jnp.clip(deg, 1.0, None)
    h = aggn @ wl_ref[...] + bl_ref[...] + x_ref[...] @ wr_ref[...]
    h = jnp.maximum(h, 0.0)
    h_ref[...] = h
    attn = attn_ref[...]
    nrm = jnp.sqrt(jnp.sum(attn * attn))
    s_ref[...] = jnp.tanh(jnp.sum(h * attn, axis=1, keepdims=True) / nrm)
    gid = lax.broadcasted_iota(jnp.int32, (1, G), 1)
    oh = jnp.where((b_ref[...] == gid) & (k_ref[...] > 0.0), 1.0, 0.0)
    cnt = jnp.sum(oh, axis=0, keepdims=True)

    @pl.when(pl.program_id(0) == 0)
    def _():
        cnt_ref[...] = jnp.zeros_like(cnt_ref)

    cnt_ref[...] += cnt


def _t1_sage(agg2, x, kv, bc, wl, bl, wr, attn):
    return _PC(
        _t1_body,
        grid=(NBLK,),
        in_specs=[
            pl.BlockSpec((NC, R, DE), lambda i: (0, i, 0)),
            pl.BlockSpec((R, D), lambda i: (i, 0)),
            pl.BlockSpec((R, 1), lambda i: (i, 0)),
            pl.BlockSpec((R, 1), lambda i: (i, 0)),
            pl.BlockSpec((D, D), lambda i: (0, 0)),
            pl.BlockSpec((1, D), lambda i: (0, 0)),
            pl.BlockSpec((D, D), lambda i: (0, 0)),
            pl.BlockSpec((1, D), lambda i: (0, 0)),
        ],
        out_specs=[
            pl.BlockSpec((R, D), lambda i: (i, 0)),
            pl.BlockSpec((R, 1), lambda i: (i, 0)),
            pl.BlockSpec((1, G), lambda i: (0, 0)),
        ],
        out_shape=[
            jax.ShapeDtypeStruct((NP, D), jnp.float32),
            jax.ShapeDtypeStruct((NP, 1), jnp.float32),
            jax.ShapeDtypeStruct((1, G), jnp.float32),
        ],
    )(agg2, x, kv, bc, wl, bl, wr, attn)


def _t23_body(h_ref, si_ref, bi_ref, ki_ref, sr_ref, br_ref, kr_ref,
              cnts_ref, bmin_ref, bmax_ref, glo_ref, ghi_ref,
              keep_ref, mx_ref, sm_ref, cnt_ref, xs_ref, xe_ref):
    p = pl.program_id(0)

    @pl.when(p == 0)
    def _():
        mx_ref[...] = jnp.full_like(mx_ref, -jnp.inf)
        sm_ref[...] = jnp.zeros_like(sm_ref)
        cnt_ref[...] = jnp.zeros_like(cnt_ref)

    si = si_ref[...]
    bi = bi_ref[...]
    vi = ki_ref[...] > 0.0
    idxi = p * R + lax.broadcasted_iota(jnp.int32, (R, 1), 0)
    gmin = bmin_ref[p * (R // G)]
    gmax = bmax_ref[p * (R // G) + (R // G) - 1]

    def jbody(j, acc):
        def hit():
            sj = sr_ref[pl.ds(j, 1), :]
            bj = br_ref[pl.ds(j, 1), :]
            vj = kr_ref[pl.ds(j, 1), :] > 0.0
            idxj = j * G + lax.broadcasted_iota(jnp.int32, (1, G), 1)
            higher = (sj > si) | ((sj == si) & (idxj < idxi))
            m = (bj == bi) & vj & higher
            return acc + jnp.sum(jnp.where(m, 1.0, 0.0), axis=1, keepdims=True)

        pred = (bmax_ref[j] >= gmin) & (bmin_ref[j] <= gmax)
        return lax.cond(pred, hit, lambda: acc)

    rank = lax.fori_loop(0, NP // G, jbody, jnp.zeros((R, 1), jnp.float32))
    kq_row = jnp.floor((4.0 * cnts_ref[...] + 4.0) / 5.0 + 2e-3)
    gid = lax.broadcasted_iota(jnp.int32, (1, G), 1)
    oh = jnp.where(bi == gid, 1.0, 0.0)
    kq = jnp.sum(oh * kq_row, axis=1, keepdims=True)
    kp = jnp.where(vi & (rank < kq), 1.0, 0.0)
    keep_ref[...] = kp

    xs = h_ref[...] * si
    xs_ref[...] = xs
    xe_ref[...] = jnp.concatenate(
        [xs * kp, kp, jnp.zeros((R, DE - D - 1), jnp.float32)], axis=1)

    def gbody(g, carry):
        m = (bi == g) & (kp > 0.0)
        mf = jnp.where(m, 1.0, 0.0)
        xm = jnp.where(m, xs, -jnp.inf)
        mx_ref[pl.ds(g, 1), :] = jnp.maximum(
            mx_ref[pl.ds(g, 1), :], jnp.max(xm, axis=0, keepdims=True))
        sm_ref[pl.ds(g, 1), :] += jnp.sum(xs * mf, axis=0, keepdims=True)
        cnt_ref[pl.ds(g, 1), :] += jnp.sum(mf).reshape(1, 1)
        return carry

    lax.fori_loop(glo_ref[p], ghi_ref[p] + 1, gbody, 0)


def _t23_pool_readout(h, score, bc, kv, sr, br, kr, counts,
                      bsmin, bsmax, gblo, gbhi):
    return _PC(
        _t23_body,
        grid=(NBLK,),
        in_specs=[
            pl.BlockSpec((R, D), lambda i: (i, 0)),
            pl.BlockSpec((R, 1), lambda i: (i, 0)),
            pl.BlockSpec((R, 1), lambda i: (i, 0)),
            pl.BlockSpec((R, 1), lambda i: (i, 0)),
            pl.BlockSpec((NP // G, G), lambda i: (0, 0)),
            pl.BlockSpec((NP // G, G), lambda i: (0, 0)),
            pl.BlockSpec((NP // G, G), lambda i: (0, 0)),
            pl.BlockSpec((1, G), lambda i: (0, 0)),
            pl.BlockSpec(memory_space=pltpu.SMEM),
            pl.BlockSpec(memory_space=pltpu.SMEM),
            pl.BlockSpec(memory_space=pltpu.SMEM),
            pl.BlockSpec(memory_space=pltpu.SMEM),
        ],
        out_specs=[
            pl.BlockSpec((R, 1), lambda i: (i, 0)),
            pl.BlockSpec((G, D), lambda i: (0, 0)),
            pl.BlockSpec((G, D), lambda i: (0, 0)),
            pl.BlockSpec((G, 1), lambda i: (0, 0)),
            pl.BlockSpec((R, D), lambda i: (i, 0)),
            pl.BlockSpec((R, DE), lambda i: (i, 0)),
        ],
        out_shape=[
            jax.ShapeDtypeStruct((NP, 1), jnp.float32),
            jax.ShapeDtypeStruct((G, D), jnp.float32),
            jax.ShapeDtypeStruct((G, D), jnp.float32),
            jax.ShapeDtypeStruct((G, 1), jnp.float32),
            jax.ShapeDtypeStruct((NP, D), jnp.float32),
            jax.ShapeDtypeStruct((NP, DE), jnp.float32),
        ],
    )(h, score, bc, kv, sr, br, kr, counts, bsmin, bsmax, gblo, gbhi)


def _t4_body(mx1, sm1, c1, mx2, sm2, c2, mx3, sm3, c3,
             wf1, bf1, g1, be1, wf2, bf2, g2, be2, wo, bo, out_ref):
    def readout(mx_ref, sm_ref, c_ref):
        cnt = c_ref[...]
        mx = jnp.where(cnt > 0.0, mx_ref[...], 0.0)
        mn = sm_ref[...] / jnp.clip(cnt, 1.0, None)
        return jnp.concatenate([mx, mn], axis=1)

    def bn(x, g_ref, be_ref):
        mu = jnp.mean(x, axis=0, keepdims=True)
        xc = x - mu
        var = jnp.mean(xc * xc, axis=0, keepdims=True)
        return g_ref[...] * xc / jnp.sqrt(var + EPS) + be_ref[...]

    h = (readout(mx1, sm1, c1) + readout(mx2, sm2, c2)
         + readout(mx3, sm3, c3))
    h = jnp.maximum(bn(h @ wf1[...] + bf1[...], g1, be1), 0.0)
    h = jnp.maximum(bn(h @ wf2[...] + bf2[...], g2, be2), 0.0)
    z = jnp.sum(h * wo[...], axis=1, keepdims=True) + bo[...]
    out_ref[...] = 1.0 / (1.0 + jnp.exp(-z))


def _t4_head(reads, p):
    args = []
    for mx, sm, cnt in reads:
        args += [mx, sm, cnt]
    args += [
        p['W_fc1'], p['b_fc1'].reshape(1, D), p['g1'].reshape(1, D),
        p['be1'].reshape(1, D),
        p['W_fc2'], p['b_fc2'].reshape(1, 64), p['g2'].reshape(1, 64),
        p['be2'].reshape(1, 64),
        p['W_out'].reshape(1, 64), p['b_out'].reshape(1, 1),
    ]
    return _PC(
        _t4_body,
        out_shape=jax.ShapeDtypeStruct((G, 1), jnp.float32),
    )(*args)


def kernel(x, edge_index, batch, params):
    p = params
    idxp = jnp.concatenate(
        [x[:, 0], jnp.zeros((NP - N,), jnp.int32)])
    bp = jnp.concatenate(
        [batch, jnp.full((NP - N,), G - 1, jnp.int32)])
    bc = bp.reshape(NP, 1)
    br = bp.reshape(NP // G, G)
    bsmin = br[:, 0]
    bsmax = br[:, -1]
    gblo = bp[::R]
    gbhi = bp[R - 1::R]
    src = edge_index[0]
    dst = edge_index[1]
    zrows = jnp.zeros((NP // NS, DE), jnp.float32)
    kv = jnp.concatenate(
        [jnp.ones((N,), jnp.float32),
         jnp.zeros((NP - N,), jnp.float32)]).reshape(NP, 1)

    xc = _sc_emb_gather(p['emb'], idxp)
    xe = jnp.concatenate(
        [xc * kv, kv, jnp.zeros((NP, DE - D - 1), jnp.float32)], axis=1)

    reads = []
    for c in (1, 2, 3):
        agg2 = _sc_edge_agg(xe, src, dst, zrows)
        h, score, counts = _t1_sage(
            agg2, xc, kv, bc, p['Wl%d' % c], p['bl%d' % c].reshape(1, D),
            p['Wr%d' % c], p['attn%d' % c].reshape(1, D))
        keep, mx, sm, cnt, xs, xe = _t23_pool_readout(
            h, score, bc, kv, score.reshape(NP // G, G), br,
            kv.reshape(NP // G, G), counts, bsmin, bsmax, gblo, gbhi)
        reads.append((mx, sm, cnt))
        xc, kv = xs, keep

    return _t4_head(reads, p).reshape(G)


# speedup vs baseline: 21.8467x; 1.3208x over previous
import functools

import jax
import jax.numpy as jnp
from jax import lax
from jax.experimental import pallas as pl
from jax.experimental.pallas import tpu as pltpu
from jax.experimental.pallas import tpu_sc as plsc

N = 10000
NP = 10240
E = 320000
G = 128
D = 128
DE = 144
R = 1024
NBLK = NP // R
NC, NS = 2, 16
NW = NC * NS
EPW = E // NW
SUP = 640
NSUP = E // SUP
RB = 3
EPS = 1e-5

_PC = pl.pallas_call


@functools.cache
def _mesh():
    return plsc.VectorSubcoreMesh(
        core_axis_name="c", subcore_axis_name="s",
        num_cores=NC, num_subcores=NS)


@functools.cache
def _make_sc_emb_gather():
    @functools.partial(
        pl.kernel,
        out_type=jax.ShapeDtypeStruct((NP, D), jnp.float32),
        mesh=_mesh(),
        scratch_types=[
            pltpu.VMEM((NP // NW,), jnp.int32),
            pltpu.VMEM((NP // NW, D), jnp.float32),
            pltpu.SemaphoreType.DMA,
        ],
    )
    def body(emb_hbm, idx_hbm, out_hbm, idx_v, rows_v, sem):
        wid = lax.axis_index("s") * NC + lax.axis_index("c")
        bpw = NP // NW
        base = pl.multiple_of(wid * bpw, bpw)
        pltpu.sync_copy(idx_hbm.at[pl.ds(base, bpw)], idx_v)
        descs = [
            pltpu.async_copy(
                emb_hbm.at[idx_v.at[pl.ds(j * 80, 80)]],
                rows_v.at[pl.ds(j * 80, 80), :], sem)
            for j in range(bpw // 80)
        ]
        for d in descs:
            d.wait()
        pltpu.sync_copy(rows_v, out_hbm.at[pl.ds(base, bpw), :])

    return body


def _sc_emb_gather(emb, idxp):
    return _make_sc_emb_gather()(emb, idxp)


@functools.cache
def _make_sc_edge_agg():
    @functools.partial(
        pl.kernel,
        out_type=jax.ShapeDtypeStruct((NC, NP, DE), jnp.float32),
        mesh=_mesh(),
        compiler_params=pltpu.CompilerParams(use_tc_tiling_on_sc=False),
        scratch_types=[
            pltpu.VMEM_SHARED((NP, DE), jnp.float32),
            pltpu.VMEM((SUP,), jnp.int32),
            pltpu.VMEM((SUP // 80, 80), jnp.int32),
            pltpu.VMEM((RB, 80, DE), jnp.float32),
            pltpu.SemaphoreType.DMA,
            pltpu.SemaphoreType.DMA,
        ],
    )
    def body(xe_hbm, src_hbm, dst_hbm, z_hbm, agg_hbm,
             shared, se_v, de_v, rows_v, sem_g, sem_s):
        c = lax.axis_index("c")
        s = lax.axis_index("s")
        wid = s * NC + c
        rps = NP // NS
        rbase = pl.multiple_of(s * rps, rps)
        pltpu.sync_copy(z_hbm, shared.at[pl.ds(rbase, rps), :])
        plsc.subcore_barrier()
        nch = SUP // 80

        def do_super(sid):
            ebase = pl.multiple_of(sid * SUP, SUP)
            pltpu.sync_copy(src_hbm.at[pl.ds(ebase, SUP)], se_v)
            for b in range(nch):
                pltpu.sync_copy(dst_hbm.at[pl.ds(ebase + b * 80, 80)],
                                de_v.at[b])

            def gather(b):
                return pltpu.async_copy(
                    xe_hbm.at[se_v.at[pl.ds(b * 80, 80)]],
                    rows_v.at[b % RB], sem_g)

            gd = {b: gather(b) for b in range(RB)}
            sd = {}
            for b in range(nch):
                gd[b].wait()
                sd[b] = pltpu.async_copy(
                    rows_v.at[b % RB], shared.at[de_v.at[b]], sem_s,
                    add=True)
                if b - (RB - 1) >= 0:
                    sd[b - (RB - 1)].wait()
                if b + RB < nch:
                    gd[b + RB] = gather(b + RB)
            for b in range(nch - (RB - 1), nch):
                if b >= 0:
                    sd[b].wait()

        def loop_body(t, carry):
            sid = t * NW + wid

            @pl.when(sid < NSUP)
            def _():
                do_super(sid)

            return carry

        lax.fori_loop(0, (NSUP + NW - 1) // NW, loop_body, 0)
        plsc.subcore_barrier()
        pltpu.sync_copy(shared.at[pl.ds(rbase, rps), :],
                        agg_hbm.at[c, pl.ds(rbase, rps), :])

    return body


def _sc_edge_agg(xe, src, dst, zrows):
    return _make_sc_edge_agg()(xe, src, dst, zrows)


def _t1_body(agg_ref, x_ref, k_ref, b_ref, wl_ref, bl_ref, wr_ref,
             attn_ref, h_ref, s_ref, cnt_ref):
    a2 = agg_ref[...]
    agg = a2[0] + a2[1]
    deg = agg[:, D:D + 1]
    aggn = agg[:, :D] / jnp.clip(deg, 1.0, None)
    h = aggn @ wl_ref[...] + bl_ref[...] + x_ref[...] @ wr_ref[...]
    h = jnp.maximum(h, 0.0)
    h_ref[...] = h
    attn = attn_ref[...]
    nrm = jnp.sqrt(jnp.sum(attn * attn))
    s_ref[...] = jnp.tanh(jnp.sum(h * attn, axis=1, keepdims=True) / nrm)
    gid = lax.broadcasted_iota(jnp.int32, (1, G), 1)
    oh = jnp.where((b_ref[...] == gid) & (k_ref[...] > 0.0), 1.0, 0.0)
    cnt = jnp.sum(oh, axis=0, keepdims=True)

    @pl.when(pl.program_id(0) == 0)
    def _():
        cnt_ref[...] = jnp.zeros_like(cnt_ref)

    cnt_ref[...] += cnt


def _t1_sage(agg2, x, kv, bc, wl, bl, wr, attn):
    return _PC(
        _t1_body,
        grid=(NBLK,),
        in_specs=[
            pl.BlockSpec((NC, R, DE), lambda i: (0, i, 0)),
            pl.BlockSpec((R, D), lambda i: (i, 0)),
            pl.BlockSpec((R, 1), lambda i: (i, 0)),
            pl.BlockSpec((R, 1), lambda i: (i, 0)),
            pl.BlockSpec((D, D), lambda i: (0, 0)),
            pl.BlockSpec((1, D), lambda i: (0, 0)),
            pl.BlockSpec((D, D), lambda i: (0, 0)),
            pl.BlockSpec((1, D), lambda i: (0, 0)),
        ],
        out_specs=[
            pl.BlockSpec((R, D), lambda i: (i, 0)),
            pl.BlockSpec((R, 1), lambda i: (i, 0)),
            pl.BlockSpec((1, G), lambda i: (0, 0)),
        ],
        out_shape=[
            jax.ShapeDtypeStruct((NP, D), jnp.float32),
            jax.ShapeDtypeStruct((NP, 1), jnp.float32),
            jax.ShapeDtypeStruct((1, G), jnp.float32),
        ],
    )(agg2, x, kv, bc, wl, bl, wr, attn)


def _t23_body(h_ref, si_ref, bi_ref, ki_ref, sr_ref, br_ref, kr_ref,
              cnts_ref, bmin_ref, bmax_ref, glo_ref, ghi_ref,
              keep_ref, mx_ref, sm_ref, cnt_ref, xs_ref, xe_ref, acc_ref):
    p = pl.program_id(0)

    @pl.when(p == 0)
    def _():
        mx_ref[...] = jnp.full_like(mx_ref, -jnp.inf)
        sm_ref[...] = jnp.zeros_like(sm_ref)
        cnt_ref[...] = jnp.zeros_like(cnt_ref)

    si = si_ref[...]
    bi = bi_ref[...]
    vi = ki_ref[...] > 0.0
    idxi = p * R + lax.broadcasted_iota(jnp.int32, (R, 1), 0)
    gmin = bmin_ref[p * (R // G)]
    gmax = bmax_ref[p * (R // G) + (R // G) - 1]

    acc_ref[...] = jnp.zeros((R, 1), jnp.float32)

    def jbody(j, carry):
        pred = (bmax_ref[j] >= gmin) & (bmin_ref[j] <= gmax)

        @pl.when(pred)
        def _():
            sj = sr_ref[pl.ds(j, 1), :]
            bj = br_ref[pl.ds(j, 1), :]
            vj = kr_ref[pl.ds(j, 1), :] > 0.0
            idxj = j * G + lax.broadcasted_iota(jnp.int32, (1, G), 1)
            higher = (sj > si) | ((sj == si) & (idxj < idxi))
            m = (bj == bi) & vj & higher
            acc_ref[...] += jnp.sum(jnp.where(m, 1.0, 0.0), axis=1,
                                    keepdims=True)

        return carry

    lax.fori_loop(0, NP // G, jbody, 0)
    rank = acc_ref[...]
    kq_row = jnp.floor((4.0 * cnts_ref[...] + 4.0) / 5.0 + 2e-3)
    gid = lax.broadcasted_iota(jnp.int32, (1, G), 1)
    oh = jnp.where(bi == gid, 1.0, 0.0)
    kq = jnp.sum(oh * kq_row, axis=1, keepdims=True)
    kp = jnp.where(vi & (rank < kq), 1.0, 0.0)
    keep_ref[...] = kp

    xs = h_ref[...] * si
    xs_ref[...] = xs
    xe_ref[...] = jnp.concatenate(
        [xs * kp, kp, jnp.zeros((R, DE - D - 1), jnp.float32)], axis=1)

    def gbody(g, carry):
        m = (bi == g) & (kp > 0.0)
        mf = jnp.where(m, 1.0, 0.0)
        xm = jnp.where(m, xs, -jnp.inf)
        mx_ref[pl.ds(g, 1), :] = jnp.maximum(
            mx_ref[pl.ds(g, 1), :], jnp.max(xm, axis=0, keepdims=True))
        sm_ref[pl.ds(g, 1), :] += jnp.sum(xs * mf, axis=0, keepdims=True)
        cnt_ref[pl.ds(g, 1), :] += jnp.sum(mf).reshape(1, 1)
        return carry

    lax.fori_loop(glo_ref[p], ghi_ref[p] + 1, gbody, 0)


def _t23_pool_readout(h, score, bc, kv, sr, br, kr, counts,
                      bsmin, bsmax, gblo, gbhi):
    return _PC(
        _t23_body,
        grid=(NBLK,),
        in_specs=[
            pl.BlockSpec((R, D), lambda i: (i, 0)),
            pl.BlockSpec((R, 1), lambda i: (i, 0)),
            pl.BlockSpec((R, 1), lambda i: (i, 0)),
            pl.BlockSpec((R, 1), lambda i: (i, 0)),
            pl.BlockSpec((NP // G, G), lambda i: (0, 0)),
            pl.BlockSpec((NP // G, G), lambda i: (0, 0)),
            pl.BlockSpec((NP // G, G), lambda i: (0, 0)),
            pl.BlockSpec((1, G), lambda i: (0, 0)),
            pl.BlockSpec(memory_space=pltpu.SMEM),
            pl.BlockSpec(memory_space=pltpu.SMEM),
            pl.BlockSpec(memory_space=pltpu.SMEM),
            pl.BlockSpec(memory_space=pltpu.SMEM),
        ],
        out_specs=[
            pl.BlockSpec((R, 1), lambda i: (i, 0)),
            pl.BlockSpec((G, D), lambda i: (0, 0)),
            pl.BlockSpec((G, D), lambda i: (0, 0)),
            pl.BlockSpec((G, 1), lambda i: (0, 0)),
            pl.BlockSpec((R, D), lambda i: (i, 0)),
            pl.BlockSpec((R, DE), lambda i: (i, 0)),
        ],
        out_shape=[
            jax.ShapeDtypeStruct((NP, 1), jnp.float32),
            jax.ShapeDtypeStruct((G, D), jnp.float32),
            jax.ShapeDtypeStruct((G, D), jnp.float32),
            jax.ShapeDtypeStruct((G, 1), jnp.float32),
            jax.ShapeDtypeStruct((NP, D), jnp.float32),
            jax.ShapeDtypeStruct((NP, DE), jnp.float32),
        ],
        scratch_shapes=[pltpu.VMEM((R, 1), jnp.float32)],
    )(h, score, bc, kv, sr, br, kr, counts, bsmin, bsmax, gblo, gbhi)


def _t4_body(mx1, sm1, c1, mx2, sm2, c2, mx3, sm3, c3,
             wf1, bf1, g1, be1, wf2, bf2, g2, be2, wo, bo, out_ref):
    def readout(mx_ref, sm_ref, c_ref):
        cnt = c_ref[...]
        mx = jnp.where(cnt > 0.0, mx_ref[...], 0.0)
        mn = sm_ref[...] / jnp.clip(cnt, 1.0, None)
        return jnp.concatenate([mx, mn], axis=1)

    def bn(x, g_ref, be_ref):
        mu = jnp.mean(x, axis=0, keepdims=True)
        xc = x - mu
        var = jnp.mean(xc * xc, axis=0, keepdims=True)
        return g_ref[...] * xc / jnp.sqrt(var + EPS) + be_ref[...]

    h = (readout(mx1, sm1, c1) + readout(mx2, sm2, c2)
         + readout(mx3, sm3, c3))
    h = jnp.maximum(bn(h @ wf1[...] + bf1[...], g1, be1), 0.0)
    h = jnp.maximum(bn(h @ wf2[...] + bf2[...], g2, be2), 0.0)
    z = jnp.sum(h * wo[...], axis=1, keepdims=True) + bo[...]
    out_ref[...] = 1.0 / (1.0 + jnp.exp(-z))


def _t4_head(reads, p):
    args = []
    for mx, sm, cnt in reads:
        args += [mx, sm, cnt]
    args += [
        p['W_fc1'], p['b_fc1'].reshape(1, D), p['g1'].reshape(1, D),
        p['be1'].reshape(1, D),
        p['W_fc2'], p['b_fc2'].reshape(1, 64), p['g2'].reshape(1, 64),
        p['be2'].reshape(1, 64),
        p['W_out'].reshape(1, 64), p['b_out'].reshape(1, 1),
    ]
    return _PC(
        _t4_body,
        out_shape=jax.ShapeDtypeStruct((G, 1), jnp.float32),
    )(*args)


def kernel(x, edge_index, batch, params):
    p = params
    idxp = jnp.concatenate(
        [x[:, 0], jnp.zeros((NP - N,), jnp.int32)])
    bp = jnp.concatenate(
        [batch, jnp.full((NP - N,), G - 1, jnp.int32)])
    bc = bp.reshape(NP, 1)
    br = bp.reshape(NP // G, G)
    bsmin = br[:, 0]
    bsmax = br[:, -1]
    gblo = bp[::R]
    gbhi = bp[R - 1::R]
    src = edge_index[0]
    dst = edge_index[1]
    zrows = jnp.zeros((NP // NS, DE), jnp.float32)
    kv = jnp.concatenate(
        [jnp.ones((N,), jnp.float32),
         jnp.zeros((NP - N,), jnp.float32)]).reshape(NP, 1)

    xc = _sc_emb_gather(p['emb'], idxp)
    xe = jnp.concatenate(
        [xc * kv, kv, jnp.zeros((NP, DE - D - 1), jnp.float32)], axis=1)

    reads = []
    for c in (1, 2, 3):
        agg2 = _sc_edge_agg(xe, src, dst, zrows)
        h, score, counts = _t1_sage(
            agg2, xc, kv, bc, p['Wl%d' % c], p['bl%d' % c].reshape(1, D),
            p['Wr%d' % c], p['attn%d' % c].reshape(1, D))
        keep, mx, sm, cnt, xs, xe = _t23_pool_readout(
            h, score, bc, kv, score.reshape(NP // G, G), br,
            kv.reshape(NP // G, G), counts, bsmin, bsmax, gblo, gbhi)
        reads.append((mx, sm, cnt))
        xc, kv = xs, keep

    return _t4_head(reads, p).reshape(G)


# speedup vs baseline: 22.1604x; 1.0144x over previous
import functools

import jax
import jax.numpy as jnp
from jax import lax
from jax.experimental import pallas as pl
from jax.experimental.pallas import tpu as pltpu
from jax.experimental.pallas import tpu_sc as plsc

N = 10000
NP = 10240
E = 320000
G = 128
D = 128
DE = 144
R = 1024
NBLK = NP // R
NC, NS = 2, 16
NW = NC * NS
EPW = E // NW
SUP = 1280
NSUP = E // SUP
RB = 3
EPS = 1e-5

_PC = pl.pallas_call


@functools.cache
def _mesh():
    return plsc.VectorSubcoreMesh(
        core_axis_name="c", subcore_axis_name="s",
        num_cores=NC, num_subcores=NS)


@functools.cache
def _make_sc_emb_gather():
    @functools.partial(
        pl.kernel,
        out_type=jax.ShapeDtypeStruct((NP, D), jnp.float32),
        mesh=_mesh(),
        scratch_types=[
            pltpu.VMEM((NP // NW,), jnp.int32),
            pltpu.VMEM((NP // NW, D), jnp.float32),
            pltpu.SemaphoreType.DMA,
        ],
    )
    def body(emb_hbm, idx_hbm, out_hbm, idx_v, rows_v, sem):
        wid = lax.axis_index("s") * NC + lax.axis_index("c")
        bpw = NP // NW
        base = pl.multiple_of(wid * bpw, bpw)
        pltpu.sync_copy(idx_hbm.at[pl.ds(base, bpw)], idx_v)
        descs = [
            pltpu.async_copy(
                emb_hbm.at[idx_v.at[pl.ds(j * 80, 80)]],
                rows_v.at[pl.ds(j * 80, 80), :], sem)
            for j in range(bpw // 80)
        ]
        for d in descs:
            d.wait()
        pltpu.sync_copy(rows_v, out_hbm.at[pl.ds(base, bpw), :])

    return body


def _sc_emb_gather(emb, idxp):
    return _make_sc_emb_gather()(emb, idxp)


@functools.cache
def _make_sc_edge_agg():
    @functools.partial(
        pl.kernel,
        out_type=jax.ShapeDtypeStruct((NC, NP, DE), jnp.float32),
        mesh=_mesh(),
        compiler_params=pltpu.CompilerParams(use_tc_tiling_on_sc=False),
        scratch_types=[
            pltpu.VMEM_SHARED((NP, DE), jnp.float32),
            pltpu.VMEM((SUP,), jnp.int32),
            pltpu.VMEM((SUP // 80, 80), jnp.int32),
            pltpu.VMEM((RB, 80, DE), jnp.float32),
            pltpu.SemaphoreType.DMA,
            pltpu.SemaphoreType.DMA,
        ],
    )
    def body(xe_hbm, src_hbm, dst_hbm, z_hbm, agg_hbm,
             shared, se_v, de_v, rows_v, sem_g, sem_s):
        c = lax.axis_index("c")
        s = lax.axis_index("s")
        wid = s * NC + c
        rps = NP // NS
        rbase = pl.multiple_of(s * rps, rps)
        pltpu.sync_copy(z_hbm, shared.at[pl.ds(rbase, rps), :])
        plsc.subcore_barrier()
        nch = SUP // 80

        def do_super(sid):
            ebase = pl.multiple_of(sid * SUP, SUP)
            pltpu.sync_copy(src_hbm.at[pl.ds(ebase, SUP)], se_v)
            for b in range(nch):
                pltpu.sync_copy(dst_hbm.at[pl.ds(ebase + b * 80, 80)],
                                de_v.at[b])

            def gather(b):
                return pltpu.async_copy(
                    xe_hbm.at[se_v.at[pl.ds(b * 80, 80)]],
                    rows_v.at[b % RB], sem_g)

            gd = {b: gather(b) for b in range(RB)}
            sd = {}
            for b in range(nch):
                gd[b].wait()
                sd[b] = pltpu.async_copy(
                    rows_v.at[b % RB], shared.at[de_v.at[b]], sem_s,
                    add=True)
                if b - (RB - 1) >= 0:
                    sd[b - (RB - 1)].wait()
                if b + RB < nch:
                    gd[b + RB] = gather(b + RB)
            for b in range(nch - (RB - 1), nch):
                if b >= 0:
                    sd[b].wait()

        def loop_body(t, carry):
            sid = t * NW + wid

            @pl.when(sid < NSUP)
            def _():
                do_super(sid)

            return carry

        lax.fori_loop(0, (NSUP + NW - 1) // NW, loop_body, 0)
        plsc.subcore_barrier()
        pltpu.sync_copy(shared.at[pl.ds(rbase, rps), :],
                        agg_hbm.at[c, pl.ds(rbase, rps), :])

    return body


def _sc_edge_agg(xe, src, dst, zrows):
    return _make_sc_edge_agg()(xe, src, dst, zrows)


def _t1_body(agg_ref, x_ref, k_ref, b_ref, wl_ref, bl_ref, wr_ref,
             attn_ref, h_ref, s_ref, cnt_ref):
    a2 = agg_ref[...]
    agg = a2[0] + a2[1]
    deg = agg[:, D:D + 1]
    aggn = agg[:, :D] / jnp.clip(deg, 1.0, None)
    h = aggn @ wl_ref[...] + bl_ref[...] + x_ref[...] @ wr_ref[...]
    h = jnp.maximum(h, 0.0)
    h_ref[...] = h
    attn = attn_ref[...]
    nrm = jnp.sqrt(jnp.sum(attn * attn))
    s_ref[...] = jnp.tanh(jnp.sum(h * attn, axis=1, keepdims=True) / nrm)
    gid = lax.broadcasted_iota(jnp.int32, (1, G), 1)
    oh = jnp.where((b_ref[...] == gid) & (k_ref[...] > 0.0), 1.0, 0.0)
    cnt = jnp.sum(oh, axis=0, keepdims=True)

    @pl.when(pl.program_id(0) == 0)
    def _():
        cnt_ref[...] = jnp.zeros_like(cnt_ref)

    cnt_ref[...] += cnt


def _t1_sage(agg2, x, kv, bc, wl, bl, wr, attn):
    return _PC(
        _t1_body,
        grid=(NBLK,),
        in_specs=[
            pl.BlockSpec((NC, R, DE), lambda i: (0, i, 0)),
            pl.BlockSpec((R, D), lambda i: (i, 0)),
            pl.BlockSpec((R, 1), lambda i: (i, 0)),
            pl.BlockSpec((R, 1), lambda i: (i, 0)),
            pl.BlockSpec((D, D), lambda i: (0, 0)),
            pl.BlockSpec((1, D), lambda i: (0, 0)),
            pl.BlockSpec((D, D), lambda i: (0, 0)),
            pl.BlockSpec((1, D), lambda i: (0, 0)),
        ],
        out_specs=[
            pl.BlockSpec((R, D), lambda i: (i, 0)),
            pl.BlockSpec((R, 1), lambda i: (i, 0)),
            pl.BlockSpec((1, G), lambda i: (0, 0)),
        ],
        out_shape=[
            jax.ShapeDtypeStruct((NP, D), jnp.float32),
            jax.ShapeDtypeStruct((NP, 1), jnp.float32),
            jax.ShapeDtypeStruct((1, G), jnp.float32),
        ],
    )(agg2, x, kv, bc, wl, bl, wr, attn)


def _t23_body(h_ref, si_ref, bi_ref, ki_ref, sr_ref, br_ref, kr_ref,
              cnts_ref, bmin_ref, bmax_ref, glo_ref, ghi_ref,
              keep_ref, mx_ref, sm_ref, cnt_ref, xs_ref, xe_ref, acc_ref):
    p = pl.program_id(0)

    @pl.when(p == 0)
    def _():
        mx_ref[...] = jnp.full_like(mx_ref, -jnp.inf)
        sm_ref[...] = jnp.zeros_like(sm_ref)
        cnt_ref[...] = jnp.zeros_like(cnt_ref)

    si = si_ref[...]
    bi = bi_ref[...]
    vi = ki_ref[...] > 0.0
    idxi = p * R + lax.broadcasted_iota(jnp.int32, (R, 1), 0)
    gmin = bmin_ref[p * (R // G)]
    gmax = bmax_ref[p * (R // G) + (R // G) - 1]

    acc_ref[...] = jnp.zeros((R, 1), jnp.float32)

    def jbody(j, carry):
        pred = (bmax_ref[j] >= gmin) & (bmin_ref[j] <= gmax)

        @pl.when(pred)
        def _():
            sj = sr_ref[pl.ds(j, 1), :]
            bj = br_ref[pl.ds(j, 1), :]
            vj = kr_ref[pl.ds(j, 1), :] > 0.0
            idxj = j * G + lax.broadcasted_iota(jnp.int32, (1, G), 1)
            higher = (sj > si) | ((sj == si) & (idxj < idxi))
            m = (bj == bi) & vj & higher
            acc_ref[...] += jnp.sum(jnp.where(m, 1.0, 0.0), axis=1,
                                    keepdims=True)

        return carry

    lax.fori_loop(0, NP // G, jbody, 0)
    rank = acc_ref[...]
    kq_row = jnp.floor((4.0 * cnts_ref[...] + 4.0) / 5.0 + 2e-3)
    gid = lax.broadcasted_iota(jnp.int32, (1, G), 1)
    oh = jnp.where(bi == gid, 1.0, 0.0)
    kq = jnp.sum(oh * kq_row, axis=1, keepdims=True)
    kp = jnp.where(vi & (rank < kq), 1.0, 0.0)
    keep_ref[...] = kp

    xs = h_ref[...] * si
    xs_ref[...] = xs
    xe_ref[...] = jnp.concatenate(
        [xs * kp, kp, jnp.zeros((R, DE - D - 1), jnp.float32)], axis=1)

    def gbody(g, carry):
        m = (bi == g) & (kp > 0.0)
        mf = jnp.where(m, 1.0, 0.0)
        xm = jnp.where(m, xs, -jnp.inf)
        mx_ref[pl.ds(g, 1), :] = jnp.maximum(
            mx_ref[pl.ds(g, 1), :], jnp.max(xm, axis=0, keepdims=True))
        sm_ref[pl.ds(g, 1), :] += jnp.sum(xs * mf, axis=0, keepdims=True)
        cnt_ref[pl.ds(g, 1), :] += jnp.sum(mf).reshape(1, 1)
        return carry

    lax.fori_loop(glo_ref[p], ghi_ref[p] + 1, gbody, 0)


def _t23_pool_readout(h, score, bc, kv, sr, br, kr, counts,
                      bsmin, bsmax, gblo, gbhi):
    return _PC(
        _t23_body,
        grid=(NBLK,),
        in_specs=[
            pl.BlockSpec((R, D), lambda i: (i, 0)),
            pl.BlockSpec((R, 1), lambda i: (i, 0)),
            pl.BlockSpec((R, 1), lambda i: (i, 0)),
            pl.BlockSpec((R, 1), lambda i: (i, 0)),
            pl.BlockSpec((NP // G, G), lambda i: (0, 0)),
            pl.BlockSpec((NP // G, G), lambda i: (0, 0)),
            pl.BlockSpec((NP // G, G), lambda i: (0, 0)),
            pl.BlockSpec((1, G), lambda i: (0, 0)),
            pl.BlockSpec(memory_space=pltpu.SMEM),
            pl.BlockSpec(memory_space=pltpu.SMEM),
            pl.BlockSpec(memory_space=pltpu.SMEM),
            pl.BlockSpec(memory_space=pltpu.SMEM),
        ],
        out_specs=[
            pl.BlockSpec((R, 1), lambda i: (i, 0)),
            pl.BlockSpec((G, D), lambda i: (0, 0)),
            pl.BlockSpec((G, D), lambda i: (0, 0)),
            pl.BlockSpec((G, 1), lambda i: (0, 0)),
            pl.BlockSpec((R, D), lambda i: (i, 0)),
            pl.BlockSpec((R, DE), lambda i: (i, 0)),
        ],
        out_shape=[
            jax.ShapeDtypeStruct((NP, 1), jnp.float32),
            jax.ShapeDtypeStruct((G, D), jnp.float32),
            jax.ShapeDtypeStruct((G, D), jnp.float32),
            jax.ShapeDtypeStruct((G, 1), jnp.float32),
            jax.ShapeDtypeStruct((NP, D), jnp.float32),
            jax.ShapeDtypeStruct((NP, DE), jnp.float32),
        ],
        scratch_shapes=[pltpu.VMEM((R, 1), jnp.float32)],
    )(h, score, bc, kv, sr, br, kr, counts, bsmin, bsmax, gblo, gbhi)


def _t4_body(mx1, sm1, c1, mx2, sm2, c2, mx3, sm3, c3,
             wf1, bf1, g1, be1, wf2, bf2, g2, be2, wo, bo, out_ref):
    def readout(mx_ref, sm_ref, c_ref):
        cnt = c_ref[...]
        mx = jnp.where(cnt > 0.0, mx_ref[...], 0.0)
        mn = sm_ref[...] / jnp.clip(cnt, 1.0, None)
        return jnp.concatenate([mx, mn], axis=1)

    def bn(x, g_ref, be_ref):
        mu = jnp.mean(x, axis=0, keepdims=True)
        xc = x - mu
        var = jnp.mean(xc * xc, axis=0, keepdims=True)
        return g_ref[...] * xc / jnp.sqrt(var + EPS) + be_ref[...]

    h = (readout(mx1, sm1, c1) + readout(mx2, sm2, c2)
         + readout(mx3, sm3, c3))
    h = jnp.maximum(bn(h @ wf1[...] + bf1[...], g1, be1), 0.0)
    h = jnp.maximum(bn(h @ wf2[...] + bf2[...], g2, be2), 0.0)
    z = jnp.sum(h * wo[...], axis=1, keepdims=True) + bo[...]
    out_ref[...] = 1.0 / (1.0 + jnp.exp(-z))


def _t4_head(reads, p):
    args = []
    for mx, sm, cnt in reads:
        args += [mx, sm, cnt]
    args += [
        p['W_fc1'], p['b_fc1'].reshape(1, D), p['g1'].reshape(1, D),
        p['be1'].reshape(1, D),
        p['W_fc2'], p['b_fc2'].reshape(1, 64), p['g2'].reshape(1, 64),
        p['be2'].reshape(1, 64),
        p['W_out'].reshape(1, 64), p['b_out'].reshape(1, 1),
    ]
    return _PC(
        _t4_body,
        out_shape=jax.ShapeDtypeStruct((G, 1), jnp.float32),
    )(*args)


def kernel(x, edge_index, batch, params):
    p = params
    idxp = jnp.concatenate(
        [x[:, 0], jnp.zeros((NP - N,), jnp.int32)])
    bp = jnp.concatenate(
        [batch, jnp.full((NP - N,), G - 1, jnp.int32)])
    bc = bp.reshape(NP, 1)
    br = bp.reshape(NP // G, G)
    bsmin = br[:, 0]
    bsmax = br[:, -1]
    gblo = bp[::R]
    gbhi = bp[R - 1::R]
    src = edge_index[0]
    dst = edge_index[1]
    zrows = jnp.zeros((NP // NS, DE), jnp.float32)
    kv = jnp.concatenate(
        [jnp.ones((N,), jnp.float32),
         jnp.zeros((NP - N,), jnp.float32)]).reshape(NP, 1)

    xc = _sc_emb_gather(p['emb'], idxp)
    xe = jnp.concatenate(
        [xc * kv, kv, jnp.zeros((NP, DE - D - 1), jnp.float32)], axis=1)

    reads = []
    for c in (1, 2, 3):
        agg2 = _sc_edge_agg(xe, src, dst, zrows)
        h, score, counts = _t1_sage(
            agg2, xc, kv, bc, p['Wl%d' % c], p['bl%d' % c].reshape(1, D),
            p['Wr%d' % c], p['attn%d' % c].reshape(1, D))
        keep, mx, sm, cnt, xs, xe = _t23_pool_readout(
            h, score, bc, kv, score.reshape(NP // G, G), br,
            kv.reshape(NP // G, G), counts, bsmin, bsmax, gblo, gbhi)
        reads.append((mx, sm, cnt))
        xc, kv = xs, keep

    return _t4_head(reads, p).reshape(G)


# speedup vs baseline: 23.3325x; 1.0529x over previous
import functools

import jax
import jax.numpy as jnp
from jax import lax
from jax.experimental import pallas as pl
from jax.experimental.pallas import tpu as pltpu
from jax.experimental.pallas import tpu_sc as plsc

N = 10000
NP = 10240
E = 320000
G = 128
D = 128
DE = 144
R = 512
NBLK = NP // R
NC, NS = 2, 16
NW = NC * NS
EPW = E // NW
SUP = 1280
NSUP = E // SUP
RB = 3
EPS = 1e-5

_PC = pl.pallas_call


@functools.cache
def _mesh():
    return plsc.VectorSubcoreMesh(
        core_axis_name="c", subcore_axis_name="s",
        num_cores=NC, num_subcores=NS)


@functools.cache
def _make_sc_emb_gather():
    @functools.partial(
        pl.kernel,
        out_type=jax.ShapeDtypeStruct((NP, D), jnp.float32),
        mesh=_mesh(),
        scratch_types=[
            pltpu.VMEM((NP // NW,), jnp.int32),
            pltpu.VMEM((NP // NW, D), jnp.float32),
            pltpu.SemaphoreType.DMA,
        ],
    )
    def body(emb_hbm, idx_hbm, out_hbm, idx_v, rows_v, sem):
        wid = lax.axis_index("s") * NC + lax.axis_index("c")
        bpw = NP // NW
        base = pl.multiple_of(wid * bpw, bpw)
        pltpu.sync_copy(idx_hbm.at[pl.ds(base, bpw)], idx_v)
        descs = [
            pltpu.async_copy(
                emb_hbm.at[idx_v.at[pl.ds(j * 80, 80)]],
                rows_v.at[pl.ds(j * 80, 80), :], sem)
            for j in range(bpw // 80)
        ]
        for d in descs:
            d.wait()
        pltpu.sync_copy(rows_v, out_hbm.at[pl.ds(base, bpw), :])

    return body


def _sc_emb_gather(emb, idxp):
    return _make_sc_emb_gather()(emb, idxp)


@functools.cache
def _make_sc_edge_agg():
    @functools.partial(
        pl.kernel,
        out_type=jax.ShapeDtypeStruct((NC, NP, DE), jnp.float32),
        mesh=_mesh(),
        compiler_params=pltpu.CompilerParams(use_tc_tiling_on_sc=False),
        scratch_types=[
            pltpu.VMEM_SHARED((NP, DE), jnp.float32),
            pltpu.VMEM((SUP,), jnp.int32),
            pltpu.VMEM((SUP // 80, 80), jnp.int32),
            pltpu.VMEM((RB, 80, DE), jnp.float32),
            pltpu.SemaphoreType.DMA,
            pltpu.SemaphoreType.DMA,
        ],
    )
    def body(xe_hbm, src_hbm, dst_hbm, z_hbm, agg_hbm,
             shared, se_v, de_v, rows_v, sem_g, sem_s):
        c = lax.axis_index("c")
        s = lax.axis_index("s")
        wid = s * NC + c
        rps = NP // NS
        rbase = pl.multiple_of(s * rps, rps)
        pltpu.sync_copy(z_hbm, shared.at[pl.ds(rbase, rps), :])
        plsc.subcore_barrier()
        nch = SUP // 80

        def do_super(sid):
            ebase = pl.multiple_of(sid * SUP, SUP)
            pltpu.sync_copy(src_hbm.at[pl.ds(ebase, SUP)], se_v)
            for b in range(nch):
                pltpu.sync_copy(dst_hbm.at[pl.ds(ebase + b * 80, 80)],
                                de_v.at[b])

            def gather(b):
                return pltpu.async_copy(
                    xe_hbm.at[se_v.at[pl.ds(b * 80, 80)]],
                    rows_v.at[b % RB], sem_g)

            gd = {b: gather(b) for b in range(RB)}
            sd = {}
            for b in range(nch):
                gd[b].wait()
                sd[b] = pltpu.async_copy(
                    rows_v.at[b % RB], shared.at[de_v.at[b]], sem_s,
                    add=True)
                if b - (RB - 1) >= 0:
                    sd[b - (RB - 1)].wait()
                if b + RB < nch:
                    gd[b + RB] = gather(b + RB)
            for b in range(nch - (RB - 1), nch):
                if b >= 0:
                    sd[b].wait()

        def loop_body(t, carry):
            sid = t * NW + wid

            @pl.when(sid < NSUP)
            def _():
                do_super(sid)

            return carry

        lax.fori_loop(0, (NSUP + NW - 1) // NW, loop_body, 0)
        plsc.subcore_barrier()
        pltpu.sync_copy(shared.at[pl.ds(rbase, rps), :],
                        agg_hbm.at[c, pl.ds(rbase, rps), :])

    return body


def _sc_edge_agg(xe, src, dst, zrows):
    return _make_sc_edge_agg()(xe, src, dst, zrows)


def _t1_body(agg_ref, x_ref, k_ref, b_ref, wl_ref, bl_ref, wr_ref,
             attn_ref, h_ref, s_ref, cnt_ref):
    a2 = agg_ref[...]
    agg = a2[0] + a2[1]
    deg = agg[:, D:D + 1]
    aggn = agg[:, :D] / jnp.clip(deg, 1.0, None)
    h = aggn @ wl_ref[...] + bl_ref[...] + x_ref[...] @ wr_ref[...]
    h = jnp.maximum(h, 0.0)
    h_ref[...] = h
    attn = attn_ref[...]
    nrm = jnp.sqrt(jnp.sum(attn * attn))
    s_ref[...] = jnp.tanh(jnp.sum(h * attn, axis=1, keepdims=True) / nrm)
    gid = lax.broadcasted_iota(jnp.int32, (1, G), 1)
    oh = jnp.where((b_ref[...] == gid) & (k_ref[...] > 0.0), 1.0, 0.0)
    cnt = jnp.sum(oh, axis=0, keepdims=True)

    @pl.when(pl.program_id(0) == 0)
    def _():
        cnt_ref[...] = jnp.zeros_like(cnt_ref)

    cnt_ref[...] += cnt


def _t1_sage(agg2, x, kv, bc, wl, bl, wr, attn):
    return _PC(
        _t1_body,
        grid=(NBLK,),
        in_specs=[
            pl.BlockSpec((NC, R, DE), lambda i: (0, i, 0)),
            pl.BlockSpec((R, D), lambda i: (i, 0)),
            pl.BlockSpec((R, 1), lambda i: (i, 0)),
            pl.BlockSpec((R, 1), lambda i: (i, 0)),
            pl.BlockSpec((D, D), lambda i: (0, 0)),
            pl.BlockSpec((1, D), lambda i: (0, 0)),
            pl.BlockSpec((D, D), lambda i: (0, 0)),
            pl.BlockSpec((1, D), lambda i: (0, 0)),
        ],
        out_specs=[
            pl.BlockSpec((R, D), lambda i: (i, 0)),
            pl.BlockSpec((R, 1), lambda i: (i, 0)),
            pl.BlockSpec((1, G), lambda i: (0, 0)),
        ],
        out_shape=[
            jax.ShapeDtypeStruct((NP, D), jnp.float32),
            jax.ShapeDtypeStruct((NP, 1), jnp.float32),
            jax.ShapeDtypeStruct((1, G), jnp.float32),
        ],
    )(agg2, x, kv, bc, wl, bl, wr, attn)


def _t23_body(h_ref, si_ref, bi_ref, ki_ref, sr_ref, br_ref, kr_ref,
              cnts_ref, bmin_ref, bmax_ref, glo_ref, ghi_ref,
              keep_ref, mx_ref, sm_ref, cnt_ref, xs_ref, xe_ref, acc_ref):
    p = pl.program_id(0)

    @pl.when(p == 0)
    def _():
        mx_ref[...] = jnp.full_like(mx_ref, -jnp.inf)
        sm_ref[...] = jnp.zeros_like(sm_ref)
        cnt_ref[...] = jnp.zeros_like(cnt_ref)

    si = si_ref[...]
    bi = bi_ref[...]
    vi = ki_ref[...] > 0.0
    idxi = p * R + lax.broadcasted_iota(jnp.int32, (R, 1), 0)
    gmin = bmin_ref[p * (R // G)]
    gmax = bmax_ref[p * (R // G) + (R // G) - 1]

    acc_ref[...] = jnp.zeros((R, 1), jnp.float32)

    def jbody(j, carry):
        pred = (bmax_ref[j] >= gmin) & (bmin_ref[j] <= gmax)

        @pl.when(pred)
        def _():
            sj = sr_ref[pl.ds(j, 1), :]
            bj = br_ref[pl.ds(j, 1), :]
            vj = kr_ref[pl.ds(j, 1), :] > 0.0
            idxj = j * G + lax.broadcasted_iota(jnp.int32, (1, G), 1)
            higher = (sj > si) | ((sj == si) & (idxj < idxi))
            m = (bj == bi) & vj & higher
            acc_ref[...] += jnp.sum(jnp.where(m, 1.0, 0.0), axis=1,
                                    keepdims=True)

        return carry

    lax.fori_loop(0, NP // G, jbody, 0)
    rank = acc_ref[...]
    kq_row = jnp.floor((4.0 * cnts_ref[...] + 4.0) / 5.0 + 2e-3)
    gid = lax.broadcasted_iota(jnp.int32, (1, G), 1)
    oh = jnp.where(bi == gid, 1.0, 0.0)
    kq = jnp.sum(oh * kq_row, axis=1, keepdims=True)
    kp = jnp.where(vi & (rank < kq), 1.0, 0.0)
    keep_ref[...] = kp

    xs = h_ref[...] * si
    xs_ref[...] = xs
    xe_ref[...] = jnp.concatenate(
        [xs * kp, kp, jnp.zeros((R, DE - D - 1), jnp.float32)], axis=1)

    def gbody(g, carry):
        m = (bi == g) & (kp > 0.0)
        mf = jnp.where(m, 1.0, 0.0)
        xm = jnp.where(m, xs, -jnp.inf)
        mx_ref[pl.ds(g, 1), :] = jnp.maximum(
            mx_ref[pl.ds(g, 1), :], jnp.max(xm, axis=0, keepdims=True))
        sm_ref[pl.ds(g, 1), :] += jnp.sum(xs * mf, axis=0, keepdims=True)
        cnt_ref[pl.ds(g, 1), :] += jnp.sum(mf).reshape(1, 1)
        return carry

    lax.fori_loop(glo_ref[p], ghi_ref[p] + 1, gbody, 0)


def _t23_pool_readout(h, score, bc, kv, sr, br, kr, counts,
                      bsmin, bsmax, gblo, gbhi):
    return _PC(
        _t23_body,
        grid=(NBLK,),
        in_specs=[
            pl.BlockSpec((R, D), lambda i: (i, 0)),
            pl.BlockSpec((R, 1), lambda i: (i, 0)),
            pl.BlockSpec((R, 1), lambda i: (i, 0)),
            pl.BlockSpec((R, 1), lambda i: (i, 0)),
            pl.BlockSpec((NP // G, G), lambda i: (0, 0)),
            pl.BlockSpec((NP // G, G), lambda i: (0, 0)),
            pl.BlockSpec((NP // G, G), lambda i: (0, 0)),
            pl.BlockSpec((1, G), lambda i: (0, 0)),
            pl.BlockSpec(memory_space=pltpu.SMEM),
            pl.BlockSpec(memory_space=pltpu.SMEM),
            pl.BlockSpec(memory_space=pltpu.SMEM),
            pl.BlockSpec(memory_space=pltpu.SMEM),
        ],
        out_specs=[
            pl.BlockSpec((R, 1), lambda i: (i, 0)),
            pl.BlockSpec((G, D), lambda i: (0, 0)),
            pl.BlockSpec((G, D), lambda i: (0, 0)),
            pl.BlockSpec((G, 1), lambda i: (0, 0)),
            pl.BlockSpec((R, D), lambda i: (i, 0)),
            pl.BlockSpec((R, DE), lambda i: (i, 0)),
        ],
        out_shape=[
            jax.ShapeDtypeStruct((NP, 1), jnp.float32),
            jax.ShapeDtypeStruct((G, D), jnp.float32),
            jax.ShapeDtypeStruct((G, D), jnp.float32),
            jax.ShapeDtypeStruct((G, 1), jnp.float32),
            jax.ShapeDtypeStruct((NP, D), jnp.float32),
            jax.ShapeDtypeStruct((NP, DE), jnp.float32),
        ],
        scratch_shapes=[pltpu.VMEM((R, 1), jnp.float32)],
    )(h, score, bc, kv, sr, br, kr, counts, bsmin, bsmax, gblo, gbhi)


def _t4_body(mx1, sm1, c1, mx2, sm2, c2, mx3, sm3, c3,
             wf1, bf1, g1, be1, wf2, bf2, g2, be2, wo, bo, out_ref):
    def readout(mx_ref, sm_ref, c_ref):
        cnt = c_ref[...]
        mx = jnp.where(cnt > 0.0, mx_ref[...], 0.0)
        mn = sm_ref[...] / jnp.clip(cnt, 1.0, None)
        return jnp.concatenate([mx, mn], axis=1)

    def bn(x, g_ref, be_ref):
        mu = jnp.mean(x, axis=0, keepdims=True)
        xc = x - mu
        var = jnp.mean(xc * xc, axis=0, keepdims=True)
        return g_ref[...] * xc / jnp.sqrt(var + EPS) + be_ref[...]

    h = (readout(mx1, sm1, c1) + readout(mx2, sm2, c2)
         + readout(mx3, sm3, c3))
    h = jnp.maximum(bn(h @ wf1[...] + bf1[...], g1, be1), 0.0)
    h = jnp.maximum(bn(h @ wf2[...] + bf2[...], g2, be2), 0.0)
    z = jnp.sum(h * wo[...], axis=1, keepdims=True) + bo[...]
    out_ref[...] = 1.0 / (1.0 + jnp.exp(-z))


def _t4_head(reads, p):
    args = []
    for mx, sm, cnt in reads:
        args += [mx, sm, cnt]
    args += [
        p['W_fc1'], p['b_fc1'].reshape(1, D), p['g1'].reshape(1, D),
        p['be1'].reshape(1, D),
        p['W_fc2'], p['b_fc2'].reshape(1, 64), p['g2'].reshape(1, 64),
        p['be2'].reshape(1, 64),
        p['W_out'].reshape(1, 64), p['b_out'].reshape(1, 1),
    ]
    return _PC(
        _t4_body,
        out_shape=jax.ShapeDtypeStruct((G, 1), jnp.float32),
    )(*args)


def kernel(x, edge_index, batch, params):
    p = params
    idxp = jnp.concatenate(
        [x[:, 0], jnp.zeros((NP - N,), jnp.int32)])
    bp = jnp.concatenate(
        [batch, jnp.full((NP - N,), G - 1, jnp.int32)])
    bc = bp.reshape(NP, 1)
    br = bp.reshape(NP // G, G)
    bsmin = br[:, 0]
    bsmax = br[:, -1]
    gblo = bp[::R]
    gbhi = bp[R - 1::R]
    src = edge_index[0]
    dst = edge_index[1]
    zrows = jnp.zeros((NP // NS, DE), jnp.float32)
    kv = jnp.concatenate(
        [jnp.ones((N,), jnp.float32),
         jnp.zeros((NP - N,), jnp.float32)]).reshape(NP, 1)

    xc = _sc_emb_gather(p['emb'], idxp)
    xe = jnp.concatenate(
        [xc * kv, kv, jnp.zeros((NP, DE - D - 1), jnp.float32)], axis=1)

    reads = []
    for c in (1, 2, 3):
        agg2 = _sc_edge_agg(xe, src, dst, zrows)
        h, score, counts = _t1_sage(
            agg2, xc, kv, bc, p['Wl%d' % c], p['bl%d' % c].reshape(1, D),
            p['Wr%d' % c], p['attn%d' % c].reshape(1, D))
        keep, mx, sm, cnt, xs, xe = _t23_pool_readout(
            h, score, bc, kv, score.reshape(NP // G, G), br,
            kv.reshape(NP // G, G), counts, bsmin, bsmax, gblo, gbhi)
        reads.append((mx, sm, cnt))
        xc, kv = xs, keep

    return _t4_head(reads, p).reshape(G)
